# speedup vs baseline: 5.1698x; 5.1698x over previous
import functools

import jax
import jax.numpy as jnp
from jax import lax
from jax.experimental import pallas as pl
from jax.experimental.pallas import tpu as pltpu
from jax.experimental.pallas import tpu_sc as plsc

EMBED_DIM = 128
NUM_CORES = 2
NUM_SUBCORES = 16
NW = NUM_CORES * NUM_SUBCORES
CHUNK = 128


@functools.lru_cache(maxsize=None)
def _make_gather(total, dim):
    per_w = total // NW
    n_chunks = per_w // CHUNK
    mesh = plsc.VectorSubcoreMesh(core_axis_name="c", subcore_axis_name="s")

    @functools.partial(
        pl.kernel,
        out_type=jax.ShapeDtypeStruct((total, dim), jnp.float32),
        mesh=mesh,
        scratch_types=[
            pltpu.VMEM((CHUNK,), jnp.int32),
            pltpu.VMEM((CHUNK, dim), jnp.float32),
            pltpu.SemaphoreType.DMA,
        ],
    )
    def gather_kernel(idx_hbm, table_hbm, out_hbm, idx_v, rows_v, sem):
        wid = lax.axis_index("s") * NUM_CORES + lax.axis_index("c")
        base = wid * per_w

        def body(i, carry):
            off = base + i * CHUNK
            pltpu.sync_copy(idx_hbm.at[pl.ds(off, CHUNK)], idx_v)
            pltpu.async_copy(table_hbm.at[idx_v], rows_v, sem).wait()
            pltpu.sync_copy(rows_v, out_hbm.at[pl.ds(off, CHUNK)])
            return carry

        lax.fori_loop(0, n_chunks, body, 0)

    return gather_kernel


def kernel(indices, table):
    batch, tokens = indices.shape
    flat = indices.reshape(-1).astype(jnp.int32)
    out = _make_gather(flat.shape[0], table.shape[1])(flat, table)
    return out.reshape(batch, tokens, table.shape[1])


# speedup vs baseline: 9.2627x; 1.7917x over previous
import functools

import jax
import jax.numpy as jnp
from jax import lax
from jax.experimental import pallas as pl
from jax.experimental.pallas import tpu as pltpu
from jax.experimental.pallas import tpu_sc as plsc

EMBED_DIM = 128
NUM_CORES = 2
NUM_SUBCORES = 16
NW = NUM_CORES * NUM_SUBCORES
CHUNK = 128


@functools.lru_cache(maxsize=None)
def _make_gather(total, dim):
    per_w = total // NW
    n_chunks = per_w // CHUNK
    n_pairs = n_chunks // 2
    mesh = plsc.VectorSubcoreMesh(core_axis_name="c", subcore_axis_name="s")

    @functools.partial(
        pl.kernel,
        out_type=jax.ShapeDtypeStruct((total, dim), jnp.float32),
        mesh=mesh,
        scratch_types=[
            pltpu.VMEM((n_chunks, CHUNK), jnp.int32),
            pltpu.VMEM((CHUNK, dim), jnp.float32),
            pltpu.VMEM((CHUNK, dim), jnp.float32),
            pltpu.SemaphoreType.DMA,
            pltpu.SemaphoreType.DMA,
            pltpu.SemaphoreType.DMA,
            pltpu.SemaphoreType.DMA,
        ],
    )
    def gather_kernel(idx_hbm, table_hbm, out_hbm, idx_v, rows_a, rows_b,
                      sem_ga, sem_gb, sem_sa, sem_sb):
        wid = lax.axis_index("s") * NUM_CORES + lax.axis_index("c")
        base = wid * per_w

        pltpu.sync_copy(idx_hbm.at[wid], idx_v)

        def gather(c, buf, sem):
            return pltpu.async_copy(table_hbm.at[idx_v.at[c]], buf, sem)

        def store(c, buf, sem):
            return pltpu.async_copy(buf, out_hbm.at[pl.ds(base + c * CHUNK, CHUNK)], sem)

        def wait_gather(buf, sem):
            pltpu.make_async_copy(table_hbm.at[idx_v.at[0]], buf, sem).wait()

        def wait_store(buf, sem):
            pltpu.make_async_copy(buf, out_hbm.at[pl.ds(base, CHUNK)], sem).wait()

        gather(0, rows_a, sem_ga)

        def body(j, carry):
            c0 = 2 * j
            c1 = c0 + 1
            gather(c1, rows_b, sem_gb)
            wait_gather(rows_a, sem_ga)
            store(c0, rows_a, sem_sa)
            wait_store(rows_a, sem_sa)
            gather(c0 + 2, rows_a, sem_ga)
            wait_gather(rows_b, sem_gb)
            store(c1, rows_b, sem_sb)
            wait_store(rows_b, sem_sb)
            return carry

        lax.fori_loop(0, n_pairs - 1, body, 0)

        c0 = n_chunks - 2
        c1 = n_chunks - 1
        gather(c1, rows_b, sem_gb)
        wait_gather(rows_a, sem_ga)
        store(c0, rows_a, sem_sa)
        wait_gather(rows_b, sem_gb)
        store(c1, rows_b, sem_sb)
        wait_store(rows_a, sem_sa)
        wait_store(rows_b, sem_sb)

    return gather_kernel


def kernel(indices, table):
    batch, tokens = indices.shape
    total = batch * tokens
    per_w = total // NW
    idx3 = indices.reshape(NW, per_w // CHUNK, CHUNK).astype(jnp.int32)
    out = _make_gather(total, table.shape[1])(idx3, table)
    return out.reshape(batch, tokens, table.shape[1])
